# speedup vs baseline: 1.3470x; 1.3470x over previous
import functools

import jax
import jax.numpy as jnp
from jax import lax
from jax.experimental import pallas as pl
from jax.experimental.pallas import tpu as pltpu
from jax.experimental.pallas import tpu_sc as plsc

_VOCAB = 49408
_N_WORDS = 77
_D = 768
_BATCH = 1024
_TOTAL = _BATCH * _N_WORDS
_NC = 2
_NS = 16
_L = 16
_NW = _NC * _NS
_ROWS_PER_W = _TOTAL // _NW
_C = 32
_NCHUNK = _ROWS_PER_W // _C

_mesh = plsc.VectorSubcoreMesh(core_axis_name="c", subcore_axis_name="s")


@functools.partial(
    pl.kernel,
    out_type=jax.ShapeDtypeStruct((_TOTAL, _D), jnp.float32),
    mesh=_mesh,
    scratch_types=[
        pltpu.VMEM((_C,), jnp.int32),
        pltpu.VMEM((_C,), jnp.int32),
        pltpu.VMEM((_C, _D), jnp.float32),
        pltpu.VMEM((_C, _D), jnp.float32),
        pltpu.SemaphoreType.DMA,
    ],
)
def _emb_kernel(ids_hbm, pids_hbm, tok_hbm, pos_hbm, out_hbm,
                idx_t, idx_p, rows_t, rows_p, sem):
    wid = lax.axis_index("s") * _NC + lax.axis_index("c")
    base = wid * _ROWS_PER_W

    @pl.loop(0, _NCHUNK)
    def _chunk(ci):
        off = base + ci * _C
        pltpu.sync_copy(ids_hbm.at[pl.ds(off, _C)], idx_t)
        pltpu.sync_copy(pids_hbm.at[pl.ds(off, _C)], idx_p)
        cp_t = pltpu.async_copy(tok_hbm.at[idx_t], rows_t, sem)
        cp_p = pltpu.async_copy(pos_hbm.at[idx_p], rows_p, sem)
        cp_t.wait()
        cp_p.wait()

        @pl.loop(0, _C)
        def _row(r):
            for j in range(_D // _L):
                sl = pl.ds(j * _L, _L)
                rows_t[r, sl] = rows_t[r, sl] + rows_p[r, sl]

        pltpu.sync_copy(rows_t, out_hbm.at[pl.ds(off, _C)])


def kernel(input_ids, pos_ids, token_table, pos_table):
    ids = input_ids.reshape(-1).astype(jnp.int32)
    pids = pos_ids.reshape(-1).astype(jnp.int32)
    out = _emb_kernel(ids, pids, token_table, pos_table)
    return out.reshape(_BATCH, _N_WORDS, _D)


# speedup vs baseline: 1.5138x; 1.1238x over previous
import functools

import jax
import jax.numpy as jnp
from jax import lax
from jax.experimental import pallas as pl
from jax.experimental.pallas import tpu as pltpu
from jax.experimental.pallas import tpu_sc as plsc

_N_WORDS = 77
_D = 768
_BATCH = 1024
_TOTAL = _BATCH * _N_WORDS
_NC = 2
_NS = 16
_L = 16
_NW = _NC * _NS
_ROWS_PER_W = _TOTAL // _NW
_C = 16
_NCHUNK = _ROWS_PER_W // _C

_mesh = plsc.VectorSubcoreMesh(core_axis_name="c", subcore_axis_name="s")


@functools.partial(
    pl.kernel,
    out_type=jax.ShapeDtypeStruct((_TOTAL, _D), jnp.float32),
    mesh=_mesh,
    scratch_types=[
        pltpu.VMEM((_C,), jnp.int32), pltpu.VMEM((_C,), jnp.int32),
        pltpu.VMEM((_C,), jnp.int32), pltpu.VMEM((_C,), jnp.int32),
        pltpu.VMEM((_C, _D), jnp.float32), pltpu.VMEM((_C, _D), jnp.float32),
        pltpu.VMEM((_C, _D), jnp.float32), pltpu.VMEM((_C, _D), jnp.float32),
        pltpu.VMEM((_C, _D), jnp.float32), pltpu.VMEM((_C, _D), jnp.float32),
        pltpu.SemaphoreType.DMA, pltpu.SemaphoreType.DMA,
        pltpu.SemaphoreType.DMA, pltpu.SemaphoreType.DMA,
    ],
)
def _emb_kernel(ids_hbm, pids_hbm, tok_hbm, pos_hbm, out_hbm,
                idx_t0, idx_t1, idx_p0, idx_p1,
                rows_t0, rows_t1, rows_p0, rows_p1, sbuf0, sbuf1,
                gsem0, gsem1, ssem0, ssem1):
    idx_t = (idx_t0, idx_t1)
    idx_p = (idx_p0, idx_p1)
    rows_t = (rows_t0, rows_t1)
    rows_p = (rows_p0, rows_p1)
    sbuf = (sbuf0, sbuf1)
    gsem = (gsem0, gsem1)
    ssem = (ssem0, ssem1)

    wid = lax.axis_index("s") * _NC + lax.axis_index("c")
    base = wid * _ROWS_PER_W

    def fire_gathers(b, off):
        pltpu.sync_copy(ids_hbm.at[pl.ds(off, _C)], idx_t[b])
        pltpu.sync_copy(pids_hbm.at[pl.ds(off, _C)], idx_p[b])
        pltpu.async_copy(tok_hbm.at[idx_t[b]], rows_t[b], gsem[b])
        pltpu.async_copy(pos_hbm.at[idx_p[b]], rows_p[b], gsem[b])

    def wait_gathers(b):
        pltpu.make_async_copy(tok_hbm.at[idx_t[b]], rows_t[b], gsem[b]).wait()
        pltpu.make_async_copy(pos_hbm.at[idx_p[b]], rows_p[b], gsem[b]).wait()

    def wait_store(b, off):
        pltpu.make_async_copy(sbuf[b], out_hbm.at[pl.ds(off, _C)], ssem[b]).wait()

    for b in (0, 1):
        fire_gathers(b, base + b * _C)

    @pl.loop(0, _NCHUNK, step=2)
    def _pair(ci):
        for b in (0, 1):
            c = ci + b
            off = base + c * _C
            wait_gathers(b)

            @pl.when(ci >= 2)
            def _():
                wait_store(b, off)

            @pl.loop(0, _C)
            def _row(r):
                for j in range(_D // _L):
                    sl = pl.ds(j * _L, _L)
                    sbuf[b][r, sl] = rows_t[b][r, sl] + rows_p[b][r, sl]

            @pl.when(c + 2 < _NCHUNK)
            def _():
                fire_gathers(b, off + 2 * _C)

            pltpu.async_copy(sbuf[b], out_hbm.at[pl.ds(off, _C)], ssem[b])

    for b in (0, 1):
        wait_store(b, base)


def kernel(input_ids, pos_ids, token_table, pos_table):
    ids = input_ids.reshape(-1).astype(jnp.int32)
    pids = pos_ids.reshape(-1).astype(jnp.int32)
    out = _emb_kernel(ids, pids, token_table, pos_table)
    return out.reshape(_BATCH, _N_WORDS, _D)


# speedup vs baseline: 2.7389x; 1.8093x over previous
import jax
import jax.numpy as jnp
from jax import lax
from jax.experimental import pallas as pl
from jax.experimental.pallas import tpu as pltpu
from jax.experimental.pallas import tpu_sc as plsc

_N_WORDS = 77
_D = 768
_BATCH = 1024
_TOTAL = _BATCH * _N_WORDS
_NC = 2
_NS = 16
_L = 16
_NW = _NC * _NS
_ROWS_PER_W = _TOTAL // _NW
_C = 16
_NCHUNK = _ROWS_PER_W // _C

_mesh = plsc.VectorSubcoreMesh(
    core_axis_name="c", subcore_axis_name="s", num_cores=_NC, num_subcores=_NS)

_SCRATCH = [
    pltpu.VMEM((_C,), jnp.int32), pltpu.VMEM((_C,), jnp.int32),
    pltpu.VMEM((_C,), jnp.int32), pltpu.VMEM((_C,), jnp.int32),
    pltpu.VMEM((_C, _D), jnp.float32), pltpu.VMEM((_C, _D), jnp.float32),
    pltpu.VMEM((_C, _D), jnp.float32), pltpu.VMEM((_C, _D), jnp.float32),
    pltpu.VMEM((_C, _D), jnp.float32), pltpu.VMEM((_C, _D), jnp.float32),
    pltpu.SemaphoreType.DMA, pltpu.SemaphoreType.DMA,
    pltpu.SemaphoreType.DMA, pltpu.SemaphoreType.DMA,
]


def _emb_body(ids_hbm, pids_hbm, tok_hbm, pos_hbm, out_hbm,
              idx_t0, idx_t1, idx_p0, idx_p1,
              rows_t0, rows_t1, rows_p0, rows_p1, sbuf0, sbuf1,
              gsem0, gsem1, ssem0, ssem1):
    idx_t = (idx_t0, idx_t1)
    idx_p = (idx_p0, idx_p1)
    rows_t = (rows_t0, rows_t1)
    rows_p = (rows_p0, rows_p1)
    sbuf = (sbuf0, sbuf1)
    gsem = (gsem0, gsem1)
    ssem = (ssem0, ssem1)

    wid = lax.axis_index("s") * _NC + lax.axis_index("c")
    base = wid * _ROWS_PER_W

    def fire_gathers(b, off):
        pltpu.sync_copy(ids_hbm.at[pl.ds(off, _C)], idx_t[b])
        pltpu.sync_copy(pids_hbm.at[pl.ds(off, _C)], idx_p[b])
        pltpu.async_copy(tok_hbm.at[idx_t[b]], rows_t[b], gsem[b])
        pltpu.async_copy(pos_hbm.at[idx_p[b]], rows_p[b], gsem[b])

    def wait_gathers(b):
        pltpu.make_async_copy(tok_hbm.at[idx_t[b]], rows_t[b], gsem[b]).wait()
        pltpu.make_async_copy(pos_hbm.at[idx_p[b]], rows_p[b], gsem[b]).wait()

    def wait_store(b, off):
        pltpu.make_async_copy(sbuf[b], out_hbm.at[pl.ds(off, _C)], ssem[b]).wait()

    for b in (0, 1):
        fire_gathers(b, base + b * _C)

    @pl.loop(0, _NCHUNK, step=2)
    def _pair(ci):
        for b in (0, 1):
            c = ci + b
            off = base + c * _C
            wait_gathers(b)

            @pl.when(ci >= 2)
            def _():
                wait_store(b, off)

            @pl.loop(0, _C)
            def _row(r):
                for j in range(_D // _L):
                    sl = pl.ds(j * _L, _L)
                    sbuf[b][r, sl] = rows_t[b][r, sl] + rows_p[b][r, sl]

            @pl.when(c + 2 < _NCHUNK)
            def _():
                fire_gathers(b, off + 2 * _C)

            pltpu.async_copy(sbuf[b], out_hbm.at[pl.ds(off, _C)], ssem[b])

    for b in (0, 1):
        wait_store(b, base)


_emb_kernel = pl.kernel(
    _emb_body,
    out_type=jax.ShapeDtypeStruct((_TOTAL, _D), jnp.float32),
    mesh=_mesh,
    scratch_types=_SCRATCH,
)


def kernel(input_ids, pos_ids, token_table, pos_table):
    ids = input_ids.astype(jnp.int32).T.reshape(-1)
    pids = pos_ids.astype(jnp.int32).T.reshape(-1)
    out = _emb_kernel(ids, pids, token_table, pos_table)
    return out.reshape(_N_WORDS, _BATCH, _D).transpose(1, 0, 2)


# speedup vs baseline: 2.9796x; 1.0879x over previous
import jax
import jax.numpy as jnp
from jax import lax
from jax.experimental import pallas as pl
from jax.experimental.pallas import tpu as pltpu
from jax.experimental.pallas import tpu_sc as plsc

_N_WORDS = 77
_D = 768
_BATCH = 1024
_TOTAL = _BATCH * _N_WORDS
_NC = 2
_NS = 16
_L = 16
_NW = _NC * _NS
_ROWS_PER_W = _TOTAL // _NW
_C = 16
_NCHUNK = _ROWS_PER_W // _C

_mesh = plsc.VectorSubcoreMesh(
    core_axis_name="c", subcore_axis_name="s", num_cores=_NC, num_subcores=_NS)

_SCRATCH = [
    pltpu.VMEM((_ROWS_PER_W,), jnp.int32), pltpu.VMEM((_ROWS_PER_W,), jnp.int32),
    pltpu.VMEM((_C, _D), jnp.float32), pltpu.VMEM((_C, _D), jnp.float32),
    pltpu.VMEM((_C, _D), jnp.float32), pltpu.VMEM((_C, _D), jnp.float32),
    pltpu.VMEM((_C, _D), jnp.float32), pltpu.VMEM((_C, _D), jnp.float32),
    pltpu.SemaphoreType.DMA, pltpu.SemaphoreType.DMA,
    pltpu.SemaphoreType.DMA, pltpu.SemaphoreType.DMA,
]


def _emb_body(ids_hbm, pids_hbm, tok_hbm, pos_hbm, out_hbm,
              idx_at, idx_ap,
              rows_t0, rows_t1, rows_p0, rows_p1, sbuf0, sbuf1,
              gsem0, gsem1, ssem0, ssem1):
    rows_t = (rows_t0, rows_t1)
    rows_p = (rows_p0, rows_p1)
    sbuf = (sbuf0, sbuf1)
    gsem = (gsem0, gsem1)
    ssem = (ssem0, ssem1)

    wid = lax.axis_index("s") * _NC + lax.axis_index("c")
    base = wid * _ROWS_PER_W

    def fire_gathers(b, loc):
        pltpu.async_copy(tok_hbm.at[idx_at.at[pl.ds(loc, _C)]], rows_t[b], gsem[b])
        pltpu.async_copy(pos_hbm.at[idx_ap.at[pl.ds(loc, _C)]], rows_p[b], gsem[b])

    def wait_gathers(b):
        pltpu.make_async_copy(tok_hbm.at[idx_at.at[pl.ds(0, _C)]], rows_t[b], gsem[b]).wait()
        pltpu.make_async_copy(pos_hbm.at[idx_ap.at[pl.ds(0, _C)]], rows_p[b], gsem[b]).wait()

    def wait_store(b, off):
        pltpu.make_async_copy(sbuf[b], out_hbm.at[pl.ds(off, _C)], ssem[b]).wait()

    pltpu.sync_copy(ids_hbm.at[pl.ds(base, _ROWS_PER_W)], idx_at)
    pltpu.sync_copy(pids_hbm.at[pl.ds(base, _ROWS_PER_W)], idx_ap)
    for b in (0, 1):
        fire_gathers(b, b * _C)

    @pl.loop(0, _NCHUNK, step=2)
    def _pair(ci):
        for b in (0, 1):
            c = ci + b
            off = base + c * _C
            wait_gathers(b)

            @pl.when(ci >= 2)
            def _():
                wait_store(b, off)

            @pl.loop(0, _C)
            def _row(r):
                for j in range(_D // _L):
                    sl = pl.ds(j * _L, _L)
                    sbuf[b][r, sl] = rows_t[b][r, sl] + rows_p[b][r, sl]

            @pl.when(c + 2 < _NCHUNK)
            def _():
                fire_gathers(b, (c + 2) * _C)

            pltpu.async_copy(sbuf[b], out_hbm.at[pl.ds(off, _C)], ssem[b])

    for b in (0, 1):
        wait_store(b, base)


_emb_kernel = pl.kernel(
    _emb_body,
    out_type=jax.ShapeDtypeStruct((_TOTAL, _D), jnp.float32),
    mesh=_mesh,
    scratch_types=_SCRATCH,
)


def kernel(input_ids, pos_ids, token_table, pos_table):
    ids = input_ids.astype(jnp.int32).T.reshape(-1)
    pids = pos_ids.astype(jnp.int32).T.reshape(-1)
    out = _emb_kernel(ids, pids, token_table, pos_table)
    return out.reshape(_N_WORDS, _BATCH, _D).transpose(1, 0, 2)
